# baseline (speedup 1.0000x reference)
import jax
import jax.numpy as jnp
from jax import lax
from jax.experimental import pallas as pl
from jax.experimental.pallas import tpu as pltpu
from jax.experimental.pallas import tpu_sc as plsc

N = 10000
NP = N + 8
E = 320000
VOCAB = 10000
D = 128
B = 16
LD = 64

NC = 2
NS = 16
NW = NC * NS

EPW = E // NW
ECH = 128
ENC = 80
EPAD = ENC * ECH - EPW
NBUF = 2
GSZ = 8
NGRP = ENC // GSZ

DB = 4
DGROUPS = ENC // DB

RCH = 40
RNC = N // RCH
RK = -(-RNC // NW)

RPT = 632
RPT_LAST = N - (NS - 1) * RPT

_mesh = plsc.VectorSubcoreMesh(core_axis_name="c", subcore_axis_name="s")


def _sc_gather_deg_body(t1_hbm, x_hbm, dstr_hbm, z128_hbm, ones_hbm,
                        hw1_hbm, hist_hbm,
                        hist_acc, xin_v, rows_v, din_v, ones_v, dsem):
  c = lax.axis_index("c")
  s = lax.axis_index("s")
  wid = c * NS + s
  r0 = s * RPT
  @pl.when(s < NS - 1)
  def _():
    pltpu.sync_copy(z128_hbm, hist_acc.at[pl.ds(r0, RPT)])
  @pl.when(s == NS - 1)
  def _():
    pltpu.sync_copy(z128_hbm.at[pl.ds(0, RPT_LAST)],
                    hist_acc.at[pl.ds(r0, RPT_LAST)])
  pltpu.sync_copy(ones_hbm, ones_v)
  pltpu.sync_copy(dstr_hbm.at[wid], din_v)
  plsc.subcore_barrier()
  @pl.loop(0, RK)
  def _(k):
    cid = wid + k * NW
    @pl.when(cid < RNC)
    def _():
      pltpu.sync_copy(x_hbm.at[pl.ds(cid * RCH, RCH)], xin_v)
      pltpu.sync_copy(t1_hbm.at[xin_v], rows_v)
      pltpu.sync_copy(rows_v, hw1_hbm.at[pl.ds(cid * RCH, RCH)])
  def _dscat(i, b):
    return pltpu.make_async_copy(ones_v, hist_acc.at[din_v.at[i]],
                                 dsem.at[b])

  @pl.loop(0, DGROUPS)
  def _(g):
    for b in range(DB):
      i = g * DB + b
      @pl.when(i >= DB)
      def _():
        _dscat(i - DB, b).wait()
      _dscat(i, b).start(add=True)
  for i in range(ENC - DB, ENC):
    _dscat(i, i % DB).wait()
  plsc.subcore_barrier()
  @pl.when(s < NS - 1)
  def _():
    pltpu.sync_copy(hist_acc.at[pl.ds(r0, RPT)],
                    hist_hbm.at[pl.ds(c * N + r0, RPT)])
  @pl.when(s == NS - 1)
  def _():
    pltpu.sync_copy(hist_acc.at[pl.ds(r0, RPT_LAST)],
                    hist_hbm.at[pl.ds(c * N + r0, RPT_LAST)])


_sc_gather_deg = pl.kernel(
    _sc_gather_deg_body,
    out_type=(jax.ShapeDtypeStruct((N, D), jnp.float32),
              jax.ShapeDtypeStruct((NC * N, D), jnp.float32)),
    mesh=_mesh,
    scratch_types=[
        pltpu.VMEM_SHARED((NP, D), jnp.float32),
        pltpu.VMEM((RCH,), jnp.int32),
        pltpu.VMEM((RCH, D), jnp.float32),
        pltpu.VMEM((ENC, ECH), jnp.int32),
        pltpu.VMEM((ECH, D), jnp.float32),
        pltpu.SemaphoreType.DMA((DB,)),
    ],
)


def _sc_edge_body(g_hbm, srcr_hbm, dstr_hbm, z128_hbm, out_hbm,
                  acc, sidx_v, didx_v, rows0, gsem, ssem):
  c = lax.axis_index("c")
  s = lax.axis_index("s")
  wid = c * NS + s
  r0 = s * RPT
  @pl.when(s < NS - 1)
  def _():
    pltpu.sync_copy(z128_hbm, acc.at[pl.ds(r0, RPT)])
  @pl.when(s == NS - 1)
  def _():
    pltpu.sync_copy(z128_hbm.at[pl.ds(0, RPT_LAST)],
                    acc.at[pl.ds(r0, RPT_LAST)])
  pltpu.sync_copy(srcr_hbm.at[wid], sidx_v)
  pltpu.sync_copy(dstr_hbm.at[wid], didx_v)
  plsc.subcore_barrier()

  @pl.loop(0, ENC)
  def _(i):
    pltpu.sync_copy(g_hbm.at[sidx_v.at[i]], rows0)
    pltpu.sync_copy(rows0, acc.at[didx_v.at[i]], add=True)
  plsc.subcore_barrier()
  @pl.when(s < NS - 1)
  def _():
    pltpu.sync_copy(acc.at[pl.ds(r0, RPT)],
                    out_hbm.at[pl.ds(c * N + r0, RPT)])
  @pl.when(s == NS - 1)
  def _():
    pltpu.sync_copy(acc.at[pl.ds(r0, RPT_LAST)],
                    out_hbm.at[pl.ds(c * N + r0, RPT_LAST)])


_sc_edge = pl.kernel(
    _sc_edge_body,
    out_type=jax.ShapeDtypeStruct((NC * N, D), jnp.float32),
    mesh=_mesh,
    scratch_types=[
        pltpu.VMEM_SHARED((NP, D), jnp.float32),
        pltpu.VMEM((ENC, ECH), jnp.int32),
        pltpu.VMEM((ENC, ECH), jnp.int32),
        pltpu.VMEM((ECH, D), jnp.float32),
        pltpu.SemaphoreType.DMA((NBUF,)),
        pltpu.SemaphoreType.DMA((NBUF,)),
    ],
)


def _tc_t1_body(emb_ref, w1_ref, o_ref):
  o_ref[...] = jnp.dot(emb_ref[...], w1_ref[...],
                       preferred_element_type=jnp.float32)


_tc_t1 = pl.pallas_call(
    _tc_t1_body,
    out_shape=jax.ShapeDtypeStruct((VOCAB, D), jnp.float32),
)


def _tc_scale_body(hw1_ref, hist_ref, g1_ref, dinv_ref):
  deg = 1.0 + hist_ref[0:N, 0:1] + hist_ref[N:2 * N, 0:1]
  dinv = lax.rsqrt(deg)
  dinv_ref[...] = dinv
  g1_ref[0:N] = hw1_ref[...] * dinv
  g1_ref[N:NP] = jnp.zeros((NP - N, D), jnp.float32)


_tc_scale = pl.pallas_call(
    _tc_scale_body,
    out_shape=(jax.ShapeDtypeStruct((NP, D), jnp.float32),
               jax.ShapeDtypeStruct((N, 1), jnp.float32)),
)


def _tc_layer2_body(s1_ref, g1_ref, dinv_ref, b1_ref, w2_ref, g2_ref):
  dinv = dinv_ref[...]
  h1 = jnp.maximum(
      dinv * (s1_ref[0:N] + s1_ref[N:2 * N] + g1_ref[0:N]) + b1_ref[...], 0.0)
  hw2 = jnp.dot(h1, w2_ref[...], preferred_element_type=jnp.float32)
  g2_ref[0:N] = dinv * hw2
  g2_ref[N:NP] = jnp.zeros((NP - N, D), jnp.float32)


_tc_layer2 = pl.pallas_call(
    _tc_layer2_body,
    out_shape=jax.ShapeDtypeStruct((NP, D), jnp.float32),
)


def _tc_final_body(s2_ref, g2_ref, dinv_ref, b2_ref, batch_ref,
                   wl1_ref, bl1_ref, wl2_ref, bl2_ref, o_ref):
  dinv = dinv_ref[...]
  h2 = dinv * (s2_ref[0:N] + s2_ref[N:2 * N] + g2_ref[0:N]) + b2_ref[...]
  iot = lax.broadcasted_iota(jnp.int32, (B, N), 0)
  bm = (jnp.broadcast_to(batch_ref[...], (B, N)) == iot).astype(jnp.float32)
  ssum = jnp.dot(bm, h2, preferred_element_type=jnp.float32)
  cnt = jnp.sum(bm, axis=1, keepdims=True)
  pooled = ssum / jnp.maximum(cnt, 1.0)
  z = jnp.maximum(
      jnp.dot(pooled, wl1_ref[...], preferred_element_type=jnp.float32)
      + bl1_ref[...], 0.0)
  t = (jnp.dot(z, wl2_ref[...], preferred_element_type=jnp.float32)
       + bl2_ref[...])
  o_ref[...] = 1.0 / (1.0 + jnp.exp(-t))


_tc_final = pl.pallas_call(
    _tc_final_body,
    out_shape=jax.ShapeDtypeStruct((B, 1), jnp.float32),
)


def kernel(x, edge_index, batch, emb_table, W1, b1, W2, b2, Wl1, bl1, Wl2, bl2):
  x = x.astype(jnp.int32)
  e0 = edge_index[0].astype(jnp.int32).reshape(NW, EPW)
  e1 = edge_index[1].astype(jnp.int32).reshape(NW, EPW)
  src = jnp.pad(e0, ((0, 0), (0, EPAD)),
                constant_values=N).reshape(NW, ENC, ECH)
  dst = jnp.pad(e1, ((0, 0), (0, EPAD)),
                constant_values=N).reshape(NW, ENC, ECH)
  z128 = jnp.zeros((RPT, D), jnp.float32)
  ones_a = jnp.ones((ECH, D), jnp.float32)

  t1 = _tc_t1(emb_table, W1)
  hw1, hist = _sc_gather_deg(t1, x, dst, z128, ones_a)
  g1, dinv = _tc_scale(hw1, hist)
  s1 = _sc_edge(g1, src, dst, z128)
  g2 = _tc_layer2(s1, g1, dinv, b1.reshape(1, D), W2)
  s2 = _sc_edge(g2, src, dst, z128)
  out = _tc_final(s2, g2, dinv, b2.reshape(1, D),
                  batch.astype(jnp.int32).reshape(1, N),
                  Wl1, bl1.reshape(1, LD), Wl2, bl2.reshape(1, 1))
  return out


# speedup vs baseline: 2.2866x; 2.2866x over previous
import jax
import jax.numpy as jnp
from jax import lax
from jax.experimental import pallas as pl
from jax.experimental.pallas import tpu as pltpu
from jax.experimental.pallas import tpu_sc as plsc

N = 10000
E = 320000
VOCAB = 10000
D = 128
B = 16
LD = 64

NC = 2
NS = 16
NW = NC * NS

EPW = E // NW
ECH = 80
ENC = EPW // ECH
NBUF = 2
GSZ = 25
NGRP = ENC // GSZ

DB = 4
DGROUPS = ENC // DB

RCH = 40
RNC = N // RCH
RK = -(-RNC // NW)

RPT = 632
RPT_LAST = N - (NS - 1) * RPT

_mesh = plsc.VectorSubcoreMesh(core_axis_name="c", subcore_axis_name="s")


def _sc_gather_deg_body(t1_hbm, x_hbm, dstr_hbm, z128_hbm, ones_hbm,
                        hw1_hbm, hist_hbm,
                        hist_acc, xin_v, rows_v, din_v, ones_v, dsem):
  c = lax.axis_index("c")
  s = lax.axis_index("s")
  wid = c * NS + s
  r0 = s * RPT
  @pl.when(s < NS - 1)
  def _():
    pltpu.sync_copy(z128_hbm, hist_acc.at[pl.ds(r0, RPT)])
  @pl.when(s == NS - 1)
  def _():
    pltpu.sync_copy(z128_hbm.at[pl.ds(0, RPT_LAST)],
                    hist_acc.at[pl.ds(r0, RPT_LAST)])
  pltpu.sync_copy(ones_hbm, ones_v)
  pltpu.sync_copy(dstr_hbm.at[wid], din_v)
  plsc.subcore_barrier()
  @pl.loop(0, RK)
  def _(k):
    cid = wid + k * NW
    @pl.when(cid < RNC)
    def _():
      pltpu.sync_copy(x_hbm.at[pl.ds(cid * RCH, RCH)], xin_v)
      pltpu.sync_copy(t1_hbm.at[xin_v], rows_v)
      pltpu.sync_copy(rows_v, hw1_hbm.at[pl.ds(cid * RCH, RCH)])
  def _dscat(i, b):
    return pltpu.make_async_copy(ones_v, hist_acc.at[din_v.at[i]],
                                 dsem.at[b])

  @pl.loop(0, DGROUPS)
  def _(g):
    for b in range(DB):
      i = g * DB + b
      @pl.when(i >= DB)
      def _():
        _dscat(i - DB, b).wait()
      _dscat(i, b).start(add=True)
  for i in range(DGROUPS * DB, ENC):
    _dscat(i - DB, i % DB).wait()
    _dscat(i, i % DB).start(add=True)
  for i in range(ENC - DB, ENC):
    _dscat(i, i % DB).wait()
  plsc.subcore_barrier()
  @pl.when(s < NS - 1)
  def _():
    pltpu.sync_copy(hist_acc.at[pl.ds(r0, RPT)],
                    hist_hbm.at[pl.ds(c * N + r0, RPT)])
  @pl.when(s == NS - 1)
  def _():
    pltpu.sync_copy(hist_acc.at[pl.ds(r0, RPT_LAST)],
                    hist_hbm.at[pl.ds(c * N + r0, RPT_LAST)])


_sc_gather_deg = pl.kernel(
    _sc_gather_deg_body,
    out_type=(jax.ShapeDtypeStruct((N, D), jnp.float32),
              jax.ShapeDtypeStruct((NC * N, D), jnp.float32)),
    mesh=_mesh,
    scratch_types=[
        pltpu.VMEM_SHARED((N, D), jnp.float32),
        pltpu.VMEM((RCH,), jnp.int32),
        pltpu.VMEM((RCH, D), jnp.float32),
        pltpu.VMEM((ENC, ECH), jnp.int32),
        pltpu.VMEM((ECH, D), jnp.float32),
        pltpu.SemaphoreType.DMA((DB,)),
    ],
)


def _sc_edge_body(g_hbm, srcr_hbm, dstr_hbm, z128_hbm, out_hbm,
                  acc, sidx_v, dbuf0, dbuf1, rows0, rows1, gsem, ssem):
  c = lax.axis_index("c")
  s = lax.axis_index("s")
  wid = c * NS + s
  r0 = s * RPT
  @pl.when(s < NS - 1)
  def _():
    pltpu.sync_copy(z128_hbm, acc.at[pl.ds(r0, RPT)])
  @pl.when(s == NS - 1)
  def _():
    pltpu.sync_copy(z128_hbm.at[pl.ds(0, RPT_LAST)],
                    acc.at[pl.ds(r0, RPT_LAST)])
  pltpu.sync_copy(srcr_hbm.at[wid], sidx_v)
  plsc.subcore_barrier()

  rows = (rows0, rows1)
  dbufs = (dbuf0, dbuf1)

  def _gather(i, b):
    return pltpu.make_async_copy(g_hbm.at[sidx_v.at[i]], rows[b], gsem.at[b])

  def _scatter(idx_ref, b):
    return pltpu.make_async_copy(rows[b], acc.at[idx_ref], ssem.at[b])

  def _chunk(i, k, dbuf, phase):
    slot = (phase + k) % NBUF
    nslot = (slot + 1) % NBUF
    _gather(i, slot).wait()
    _scatter(dbuf.at[k], slot).start(add=True)
    nxt = i + 1
    @pl.when(nxt < ENC)
    def _():
      @pl.when(nxt >= NBUF)
      def _():
        _scatter(dbuf.at[k], nslot).wait()
      _gather(nxt, nslot).start()

  _gather(0, 0).start()

  @pl.loop(0, (NGRP - 1) // 2)
  def _(t):
    for half in range(2):
      grp = t * 2 + half
      dbuf = dbufs[half]
      pltpu.sync_copy(dstr_hbm.at[wid, grp], dbuf)
      for k in range(GSZ):
        _chunk(grp * GSZ + k, k, dbuf, half * GSZ)

  grp = NGRP - 1
  pltpu.sync_copy(dstr_hbm.at[wid, grp], dbuf0)
  for k in range(GSZ):
    _chunk(grp * GSZ + k, k, dbuf0, 0)

  for i in range(ENC - NBUF, ENC):
    _scatter(dbuf0.at[GSZ - 1], i % NBUF).wait()
  plsc.subcore_barrier()
  @pl.when(s < NS - 1)
  def _():
    pltpu.sync_copy(acc.at[pl.ds(r0, RPT)],
                    out_hbm.at[pl.ds(c * N + r0, RPT)])
  @pl.when(s == NS - 1)
  def _():
    pltpu.sync_copy(acc.at[pl.ds(r0, RPT_LAST)],
                    out_hbm.at[pl.ds(c * N + r0, RPT_LAST)])


_sc_edge = pl.kernel(
    _sc_edge_body,
    out_type=jax.ShapeDtypeStruct((NC * N, D), jnp.float32),
    mesh=_mesh,
    scratch_types=[
        pltpu.VMEM_SHARED((N, D), jnp.float32),
        pltpu.VMEM((ENC, ECH), jnp.int32),
        pltpu.VMEM((GSZ, ECH), jnp.int32),
        pltpu.VMEM((GSZ, ECH), jnp.int32),
        pltpu.VMEM((ECH, D), jnp.float32),
        pltpu.VMEM((ECH, D), jnp.float32),
        pltpu.SemaphoreType.DMA((NBUF,)),
        pltpu.SemaphoreType.DMA((NBUF,)),
    ],
)


def _tc_t1_body(emb_ref, w1_ref, o_ref):
  o_ref[...] = jnp.dot(emb_ref[...], w1_ref[...],
                       preferred_element_type=jnp.float32)


_tc_t1 = pl.pallas_call(
    _tc_t1_body,
    out_shape=jax.ShapeDtypeStruct((VOCAB, D), jnp.float32),
)


def _tc_scale_body(hw1_ref, hist_ref, g1_ref, dinv_ref):
  deg = 1.0 + hist_ref[0:N, 0:1] + hist_ref[N:2 * N, 0:1]
  dinv = lax.rsqrt(deg)
  dinv_ref[...] = dinv
  g1_ref[...] = hw1_ref[...] * dinv


_tc_scale = pl.pallas_call(
    _tc_scale_body,
    out_shape=(jax.ShapeDtypeStruct((N, D), jnp.float32),
               jax.ShapeDtypeStruct((N, 1), jnp.float32)),
)


def _tc_layer2_body(s1_ref, g1_ref, dinv_ref, b1_ref, w2_ref, g2_ref):
  dinv = dinv_ref[...]
  h1 = jnp.maximum(
      dinv * (s1_ref[0:N] + s1_ref[N:2 * N] + g1_ref[...]) + b1_ref[...], 0.0)
  hw2 = jnp.dot(h1, w2_ref[...], preferred_element_type=jnp.float32)
  g2_ref[...] = dinv * hw2


_tc_layer2 = pl.pallas_call(
    _tc_layer2_body,
    out_shape=jax.ShapeDtypeStruct((N, D), jnp.float32),
)


def _tc_final_body(s2_ref, g2_ref, dinv_ref, b2_ref, batch_ref,
                   wl1_ref, bl1_ref, wl2_ref, bl2_ref, o_ref):
  dinv = dinv_ref[...]
  h2 = dinv * (s2_ref[0:N] + s2_ref[N:2 * N] + g2_ref[...]) + b2_ref[...]
  iot = lax.broadcasted_iota(jnp.int32, (B, N), 0)
  bm = (jnp.broadcast_to(batch_ref[...], (B, N)) == iot).astype(jnp.float32)
  ssum = jnp.dot(bm, h2, preferred_element_type=jnp.float32)
  cnt = jnp.sum(bm, axis=1, keepdims=True)
  pooled = ssum / jnp.maximum(cnt, 1.0)
  z = jnp.maximum(
      jnp.dot(pooled, wl1_ref[...], preferred_element_type=jnp.float32)
      + bl1_ref[...], 0.0)
  t = (jnp.dot(z, wl2_ref[...], preferred_element_type=jnp.float32)
       + bl2_ref[...])
  o_ref[...] = 1.0 / (1.0 + jnp.exp(-t))


_tc_final = pl.pallas_call(
    _tc_final_body,
    out_shape=jax.ShapeDtypeStruct((B, 1), jnp.float32),
)


def kernel(x, edge_index, batch, emb_table, W1, b1, W2, b2, Wl1, bl1, Wl2, bl2):
  x = x.astype(jnp.int32)
  src = edge_index[0].astype(jnp.int32).reshape(NW, ENC, ECH)
  dst = edge_index[1].astype(jnp.int32).reshape(NW, ENC, ECH)
  z128 = jnp.zeros((RPT, D), jnp.float32)
  ones_a = jnp.ones((ECH, D), jnp.float32)

  t1 = _tc_t1(emb_table, W1)
  hw1, hist = _sc_gather_deg(t1, x, dst, z128, ones_a)
  g1, dinv = _tc_scale(hw1, hist)
  dst_g = dst.reshape(NW, NGRP, GSZ, ECH)
  s1 = _sc_edge(g1, src, dst_g, z128)
  g2 = _tc_layer2(s1, g1, dinv, b1.reshape(1, D), W2)
  s2 = _sc_edge(g2, src, dst_g, z128)
  out = _tc_final(s2, g2, dinv, b2.reshape(1, D),
                  batch.astype(jnp.int32).reshape(1, N),
                  Wl1, bl1.reshape(1, LD), Wl2, bl2.reshape(1, 1))
  return out


# speedup vs baseline: 2.6216x; 1.1465x over previous
import dataclasses

import jax
import jax.numpy as jnp
from jax import lax
from jax.experimental import pallas as pl
from jax.experimental.pallas import tpu as pltpu
from jax.experimental.pallas import tpu_sc as plsc

N = 10000
E = 320000
VOCAB = 10000
D = 128
B = 16
LD = 64

NC = 2
NS = 16
NW = NC * NS

EPW = E // NW
ECH = 80
ENC = EPW // ECH
NBUF = 2
GSZ = 25
NGRP = ENC // GSZ

DB = 4
DGROUPS = ENC // DB

RCH = 40
RNC = N // RCH
RK = -(-RNC // NW)

RPT = 632
RPT_LAST = N - (NS - 1) * RPT

_mesh = plsc.VectorSubcoreMesh(core_axis_name="c", subcore_axis_name="s")

_cp = pltpu.CompilerParams()
if "needs_layout_passes" in pltpu.CompilerParams.__dataclass_fields__:
  _cp = dataclasses.replace(_cp, needs_layout_passes=False)


def _sc_gather_deg_body(t1_hbm, x_hbm, dstr_hbm, z1_hbm,
                        hw1_hbm, hist_hbm,
                        lhist, xin_v, rows_v, din_v):
  c = lax.axis_index("c")
  s = lax.axis_index("s")
  wid = c * NS + s
  pltpu.sync_copy(z1_hbm, lhist)
  pltpu.sync_copy(dstr_hbm.at[wid], din_v)
  @pl.loop(0, RK)
  def _(k):
    cid = wid + k * NW
    @pl.when(cid < RNC)
    def _():
      pltpu.sync_copy(x_hbm.at[pl.ds(cid * RCH, RCH)], xin_v)
      pltpu.sync_copy(t1_hbm.at[xin_v], rows_v)
      pltpu.sync_copy(rows_v, hw1_hbm.at[pl.ds(cid * RCH, RCH)])
  ones16 = jnp.ones((16,), jnp.float32)
  @pl.loop(0, ENC)
  def _(i):
    for v in range(ECH // 16):
      iv = din_v[i, pl.ds(v * 16, 16)]
      plsc.addupdate_scatter(lhist, [iv], ones16)
  pltpu.sync_copy(lhist, hist_hbm.at[pl.ds(wid * N, N)])


_sc_gather_deg = pl.kernel(
    _sc_gather_deg_body,
    out_type=(jax.ShapeDtypeStruct((N, D), jnp.float32),
              jax.ShapeDtypeStruct((NW * N,), jnp.float32)),
    mesh=_mesh,
    compiler_params=_cp,
    scratch_types=[
        pltpu.VMEM((N,), jnp.float32),
        pltpu.VMEM((RCH,), jnp.int32),
        pltpu.VMEM((RCH, D), jnp.float32),
        pltpu.VMEM((ENC, ECH), jnp.int32),
    ],
)


def _sc_edge_body(g_hbm, srcr_hbm, dstr_hbm, z128_hbm, out_hbm,
                  acc, sidx_v, dbuf0, dbuf1, rows0, rows1, gsem, ssem):
  c = lax.axis_index("c")
  s = lax.axis_index("s")
  wid = c * NS + s
  r0 = s * RPT
  @pl.when(s < NS - 1)
  def _():
    pltpu.sync_copy(z128_hbm, acc.at[pl.ds(r0, RPT)])
  @pl.when(s == NS - 1)
  def _():
    pltpu.sync_copy(z128_hbm.at[pl.ds(0, RPT_LAST)],
                    acc.at[pl.ds(r0, RPT_LAST)])
  pltpu.sync_copy(srcr_hbm.at[wid], sidx_v)
  plsc.subcore_barrier()

  rows = (rows0, rows1)
  dbufs = (dbuf0, dbuf1)

  def _gather(i, b):
    return pltpu.make_async_copy(g_hbm.at[sidx_v.at[i]], rows[b], gsem.at[b])

  def _scatter(idx_ref, b):
    return pltpu.make_async_copy(rows[b], acc.at[idx_ref], ssem.at[b])

  def _chunk(i, k, dbuf, phase):
    slot = (phase + k) % NBUF
    nslot = (slot + 1) % NBUF
    _gather(i, slot).wait()
    _scatter(dbuf.at[k], slot).start(add=True)
    nxt = i + 1
    @pl.when(nxt < ENC)
    def _():
      @pl.when(nxt >= NBUF)
      def _():
        _scatter(dbuf.at[k], nslot).wait()
      _gather(nxt, nslot).start()

  _gather(0, 0).start()

  @pl.loop(0, (NGRP - 1) // 2)
  def _(t):
    for half in range(2):
      grp = t * 2 + half
      dbuf = dbufs[half]
      pltpu.sync_copy(dstr_hbm.at[wid, grp], dbuf)
      for k in range(GSZ):
        _chunk(grp * GSZ + k, k, dbuf, half * GSZ)

  grp = NGRP - 1
  pltpu.sync_copy(dstr_hbm.at[wid, grp], dbuf0)
  for k in range(GSZ):
    _chunk(grp * GSZ + k, k, dbuf0, 0)

  for i in range(ENC - NBUF, ENC):
    _scatter(dbuf0.at[GSZ - 1], i % NBUF).wait()
  plsc.subcore_barrier()
  @pl.when(s < NS - 1)
  def _():
    pltpu.sync_copy(acc.at[pl.ds(r0, RPT)],
                    out_hbm.at[pl.ds(c * N + r0, RPT)])
  @pl.when(s == NS - 1)
  def _():
    pltpu.sync_copy(acc.at[pl.ds(r0, RPT_LAST)],
                    out_hbm.at[pl.ds(c * N + r0, RPT_LAST)])


_sc_edge = pl.kernel(
    _sc_edge_body,
    out_type=jax.ShapeDtypeStruct((NC * N, D), jnp.float32),
    mesh=_mesh,
    scratch_types=[
        pltpu.VMEM_SHARED((N, D), jnp.float32),
        pltpu.VMEM((ENC, ECH), jnp.int32),
        pltpu.VMEM((GSZ, ECH), jnp.int32),
        pltpu.VMEM((GSZ, ECH), jnp.int32),
        pltpu.VMEM((ECH, D), jnp.float32),
        pltpu.VMEM((ECH, D), jnp.float32),
        pltpu.SemaphoreType.DMA((NBUF,)),
        pltpu.SemaphoreType.DMA((NBUF,)),
    ],
)


def _tc_t1_body(emb_ref, w1_ref, o_ref):
  o_ref[...] = jnp.dot(emb_ref[...], w1_ref[...],
                       preferred_element_type=jnp.float32)


_tc_t1 = pl.pallas_call(
    _tc_t1_body,
    out_shape=jax.ShapeDtypeStruct((VOCAB, D), jnp.float32),
)


def _tc_scale_body(hw1_ref, hist_ref, g1_ref, dinv_ref):
  deg = 1.0 + jnp.sum(hist_ref[...], axis=0)[:, None]
  dinv = lax.rsqrt(deg)
  dinv_ref[...] = dinv
  g1_ref[...] = hw1_ref[...] * dinv


_tc_scale = pl.pallas_call(
    _tc_scale_body,
    out_shape=(jax.ShapeDtypeStruct((N, D), jnp.float32),
               jax.ShapeDtypeStruct((N, 1), jnp.float32)),
)


def _tc_layer2_body(s1_ref, g1_ref, dinv_ref, b1_ref, w2_ref, g2_ref):
  dinv = dinv_ref[...]
  h1 = jnp.maximum(
      dinv * (s1_ref[0:N] + s1_ref[N:2 * N] + g1_ref[...]) + b1_ref[...], 0.0)
  hw2 = jnp.dot(h1, w2_ref[...], preferred_element_type=jnp.float32)
  g2_ref[...] = dinv * hw2


_tc_layer2 = pl.pallas_call(
    _tc_layer2_body,
    out_shape=jax.ShapeDtypeStruct((N, D), jnp.float32),
)


def _tc_final_body(s2_ref, g2_ref, dinv_ref, b2_ref, batch_ref,
                   wl1_ref, bl1_ref, wl2_ref, bl2_ref, o_ref):
  dinv = dinv_ref[...]
  h2 = dinv * (s2_ref[0:N] + s2_ref[N:2 * N] + g2_ref[...]) + b2_ref[...]
  iot = lax.broadcasted_iota(jnp.int32, (B, N), 0)
  bm = (jnp.broadcast_to(batch_ref[...], (B, N)) == iot).astype(jnp.float32)
  ssum = jnp.dot(bm, h2, preferred_element_type=jnp.float32)
  cnt = jnp.sum(bm, axis=1, keepdims=True)
  pooled = ssum / jnp.maximum(cnt, 1.0)
  z = jnp.maximum(
      jnp.dot(pooled, wl1_ref[...], preferred_element_type=jnp.float32)
      + bl1_ref[...], 0.0)
  t = (jnp.dot(z, wl2_ref[...], preferred_element_type=jnp.float32)
       + bl2_ref[...])
  o_ref[...] = 1.0 / (1.0 + jnp.exp(-t))


_tc_final = pl.pallas_call(
    _tc_final_body,
    out_shape=jax.ShapeDtypeStruct((B, 1), jnp.float32),
)


def kernel(x, edge_index, batch, emb_table, W1, b1, W2, b2, Wl1, bl1, Wl2, bl2):
  x = x.astype(jnp.int32)
  src = edge_index[0].astype(jnp.int32).reshape(NW, ENC, ECH)
  dst = edge_index[1].astype(jnp.int32).reshape(NW, ENC, ECH)
  z128 = jnp.zeros((RPT, D), jnp.float32)
  z1d = jnp.zeros((N,), jnp.float32)

  t1 = _tc_t1(emb_table, W1)
  hw1, hist = _sc_gather_deg(t1, x, dst, z1d)
  g1, dinv = _tc_scale(hw1, hist.reshape(NW, N))
  dst_g = dst.reshape(NW, NGRP, GSZ, ECH)
  s1 = _sc_edge(g1, src, dst_g, z128)
  g2 = _tc_layer2(s1, g1, dinv, b1.reshape(1, D), W2)
  s2 = _sc_edge(g2, src, dst_g, z128)
  out = _tc_final(s2, g2, dinv, b2.reshape(1, D),
                  batch.astype(jnp.int32).reshape(1, N),
                  Wl1, bl1.reshape(1, LD), Wl2, bl2.reshape(1, 1))
  return out


# speedup vs baseline: 2.6614x; 1.0152x over previous
import dataclasses

import jax
import jax.numpy as jnp
from jax import lax
from jax.experimental import pallas as pl
from jax.experimental.pallas import tpu as pltpu
from jax.experimental.pallas import tpu_sc as plsc

N = 10000
E = 320000
VOCAB = 10000
D = 128
B = 16
LD = 64

NC = 2
NS = 16
NW = NC * NS

EPW = E // NW
ECH = 80
ENC = EPW // ECH
NBUF = 2
GSZ = 25
NGRP = ENC // GSZ

DB = 4
DGROUPS = ENC // DB

RCH = 40
RNC = N // RCH
RK = -(-RNC // NW)

RPT = 632
RPT_LAST = N - (NS - 1) * RPT

_mesh = plsc.VectorSubcoreMesh(core_axis_name="c", subcore_axis_name="s")

_cp = pltpu.CompilerParams()
if "needs_layout_passes" in pltpu.CompilerParams.__dataclass_fields__:
  _cp = dataclasses.replace(_cp, needs_layout_passes=False)


def _sc_gather_deg_body(t1_hbm, x_hbm, dstr_hbm, z1_hbm,
                        hw1_hbm, hist_hbm,
                        lhist, xin_v, rows_v, din_v):
  c = lax.axis_index("c")
  s = lax.axis_index("s")
  wid = c * NS + s
  pltpu.sync_copy(z1_hbm, lhist)
  pltpu.sync_copy(dstr_hbm.at[wid], din_v)
  @pl.loop(0, RK)
  def _(k):
    cid = wid + k * NW
    @pl.when(cid < RNC)
    def _():
      pltpu.sync_copy(x_hbm.at[pl.ds(cid * RCH, RCH)], xin_v)
      pltpu.sync_copy(t1_hbm.at[xin_v], rows_v)
      pltpu.sync_copy(rows_v, hw1_hbm.at[pl.ds(cid * RCH, RCH)])
  ones16 = jnp.ones((16,), jnp.float32)
  @pl.loop(0, ENC)
  def _(i):
    for v in range(ECH // 16):
      iv = din_v[i, pl.ds(v * 16, 16)]
      plsc.addupdate_scatter(lhist, [iv], ones16)
  pltpu.sync_copy(lhist, hist_hbm.at[pl.ds(wid * N, N)])


_sc_gather_deg = pl.kernel(
    _sc_gather_deg_body,
    out_type=(jax.ShapeDtypeStruct((N, D), jnp.float32),
              jax.ShapeDtypeStruct((NW * N,), jnp.float32)),
    mesh=_mesh,
    compiler_params=_cp,
    scratch_types=[
        pltpu.VMEM((N,), jnp.float32),
        pltpu.VMEM((RCH,), jnp.int32),
        pltpu.VMEM((RCH, D), jnp.float32),
        pltpu.VMEM((ENC, ECH), jnp.int32),
    ],
)


def _sc_edge_body(g_hbm, srcr_hbm, dstr_hbm, z128_hbm, out_hbm,
                  acc, sidx_v, dbuf0, dbuf1, rows0, rows1, gsem, ssem):
  c = lax.axis_index("c")
  s = lax.axis_index("s")
  wid = c * NS + s
  r0 = s * RPT
  @pl.when(s < NS - 1)
  def _():
    pltpu.sync_copy(z128_hbm, acc.at[pl.ds(r0, RPT)])
  @pl.when(s == NS - 1)
  def _():
    pltpu.sync_copy(z128_hbm.at[pl.ds(0, RPT_LAST)],
                    acc.at[pl.ds(r0, RPT_LAST)])
  pltpu.sync_copy(srcr_hbm.at[wid], sidx_v)
  plsc.subcore_barrier()

  rows = (rows0, rows1)
  dbufs = (dbuf0, dbuf1)

  def _gather(i, b):
    return pltpu.make_async_copy(g_hbm.at[sidx_v.at[i]], rows[b], gsem.at[b])

  def _scatter(idx_ref, b):
    return pltpu.make_async_copy(rows[b], acc.at[idx_ref], ssem.at[b])

  def _chunk(i, k, dbuf, phase):
    slot = (phase + k) % NBUF
    nslot = (slot + 1) % NBUF
    _gather(i, slot).wait()
    _scatter(dbuf.at[k], slot).start(add=True)
    nxt = i + 1
    @pl.when(nxt < ENC)
    def _():
      @pl.when(nxt >= NBUF)
      def _():
        _scatter(dbuf.at[k], nslot).wait()
      _gather(nxt, nslot).start()

  _gather(0, 0).start()

  @pl.loop(0, (NGRP - 1) // 2)
  def _(t):
    for half in range(2):
      grp = t * 2 + half
      dbuf = dbufs[half]
      pltpu.sync_copy(dstr_hbm.at[wid, grp], dbuf)
      for k in range(GSZ):
        _chunk(grp * GSZ + k, k, dbuf, half * GSZ)

  grp = NGRP - 1
  pltpu.sync_copy(dstr_hbm.at[wid, grp], dbuf0)
  for k in range(GSZ):
    _chunk(grp * GSZ + k, k, dbuf0, 0)

  for i in range(ENC - NBUF, ENC):
    _scatter(dbuf0.at[GSZ - 1], i % NBUF).wait()
  plsc.subcore_barrier()
  @pl.when(s < NS - 1)
  def _():
    pltpu.sync_copy(acc.at[pl.ds(r0, RPT)],
                    out_hbm.at[pl.ds(c * N + r0, RPT)])
  @pl.when(s == NS - 1)
  def _():
    pltpu.sync_copy(acc.at[pl.ds(r0, RPT_LAST)],
                    out_hbm.at[pl.ds(c * N + r0, RPT_LAST)])


_sc_edge = pl.kernel(
    _sc_edge_body,
    out_type=jax.ShapeDtypeStruct((NC * N, D), jnp.float32),
    mesh=_mesh,
    scratch_types=[
        pltpu.VMEM_SHARED((N, D), jnp.float32),
        pltpu.VMEM((ENC, ECH), jnp.int32),
        pltpu.VMEM((GSZ, ECH), jnp.int32),
        pltpu.VMEM((GSZ, ECH), jnp.int32),
        pltpu.VMEM((ECH, D), jnp.float32),
        pltpu.VMEM((ECH, D), jnp.float32),
        pltpu.SemaphoreType.DMA((NBUF,)),
        pltpu.SemaphoreType.DMA((NBUF,)),
    ],
)


def _tc_scale_body(h_ref, w1_ref, hist_ref, g1_ref, dinv_ref):
  deg = 1.0 + jnp.sum(hist_ref[...], axis=0)[:, None]
  dinv = lax.rsqrt(deg)
  dinv_ref[...] = dinv
  hw1 = jnp.dot(h_ref[...], w1_ref[...], preferred_element_type=jnp.float32)
  g1_ref[...] = hw1 * dinv


_tc_scale = pl.pallas_call(
    _tc_scale_body,
    out_shape=(jax.ShapeDtypeStruct((N, D), jnp.float32),
               jax.ShapeDtypeStruct((N, 1), jnp.float32)),
)


def _tc_layer2_body(s1_ref, g1_ref, dinv_ref, b1_ref, w2_ref, g2_ref):
  dinv = dinv_ref[...]
  h1 = jnp.maximum(
      dinv * (s1_ref[0:N] + s1_ref[N:2 * N] + g1_ref[...]) + b1_ref[...], 0.0)
  hw2 = jnp.dot(h1, w2_ref[...], preferred_element_type=jnp.float32)
  g2_ref[...] = dinv * hw2


_tc_layer2 = pl.pallas_call(
    _tc_layer2_body,
    out_shape=jax.ShapeDtypeStruct((N, D), jnp.float32),
)


def _tc_final_body(s2_ref, g2_ref, dinv_ref, b2_ref, batch_ref,
                   wl1_ref, bl1_ref, wl2_ref, bl2_ref, o_ref):
  dinv = dinv_ref[...]
  h2 = dinv * (s2_ref[0:N] + s2_ref[N:2 * N] + g2_ref[...]) + b2_ref[...]
  iot = lax.broadcasted_iota(jnp.int32, (B, N), 0)
  bm = (jnp.broadcast_to(batch_ref[...], (B, N)) == iot).astype(jnp.float32)
  ssum = jnp.dot(bm, h2, preferred_element_type=jnp.float32)
  cnt = jnp.sum(bm, axis=1, keepdims=True)
  pooled = ssum / jnp.maximum(cnt, 1.0)
  z = jnp.maximum(
      jnp.dot(pooled, wl1_ref[...], preferred_element_type=jnp.float32)
      + bl1_ref[...], 0.0)
  t = (jnp.dot(z, wl2_ref[...], preferred_element_type=jnp.float32)
       + bl2_ref[...])
  o_ref[...] = 1.0 / (1.0 + jnp.exp(-t))


_tc_final = pl.pallas_call(
    _tc_final_body,
    out_shape=jax.ShapeDtypeStruct((B, 1), jnp.float32),
)


def kernel(x, edge_index, batch, emb_table, W1, b1, W2, b2, Wl1, bl1, Wl2, bl2):
  x = x.astype(jnp.int32)
  src = edge_index[0].astype(jnp.int32).reshape(NW, ENC, ECH)
  dst = edge_index[1].astype(jnp.int32).reshape(NW, ENC, ECH)
  z128 = jnp.zeros((RPT, D), jnp.float32)
  z1d = jnp.zeros((N,), jnp.float32)

  h0, hist = _sc_gather_deg(emb_table, x, dst, z1d)
  g1, dinv = _tc_scale(h0, W1, hist.reshape(NW, N))
  dst_g = dst.reshape(NW, NGRP, GSZ, ECH)
  s1 = _sc_edge(g1, src, dst_g, z128)
  g2 = _tc_layer2(s1, g1, dinv, b1.reshape(1, D), W2)
  s2 = _sc_edge(g2, src, dst_g, z128)
  out = _tc_final(s2, g2, dinv, b2.reshape(1, D),
                  batch.astype(jnp.int32).reshape(1, N),
                  Wl1, bl1.reshape(1, LD), Wl2, bl2.reshape(1, 1))
  return out


# speedup vs baseline: 2.6639x; 1.0009x over previous
import dataclasses

import jax
import jax.numpy as jnp
from jax import lax
from jax.experimental import pallas as pl
from jax.experimental.pallas import tpu as pltpu
from jax.experimental.pallas import tpu_sc as plsc

N = 10000
E = 320000
VOCAB = 10000
D = 128
B = 16
LD = 64

NC = 2
NS = 16
NW = NC * NS

EPW = E // NW
ECH = 80
ENC = EPW // ECH
NBUF = 2
GSZ = 25
NGRP = ENC // GSZ

RCH = 40
RNC = N // RCH
RK = -(-RNC // NW)

RPT = 632
RPT_LAST = N - (NS - 1) * RPT

_mesh = plsc.VectorSubcoreMesh(core_axis_name="c", subcore_axis_name="s")

_cp = pltpu.CompilerParams()
if "needs_layout_passes" in pltpu.CompilerParams.__dataclass_fields__:
  _cp = dataclasses.replace(_cp, needs_layout_passes=False)


def _sc_gather_deg_body(t1_hbm, x_hbm, dstr_hbm, z1_hbm,
                        hw1_hbm, hist_hbm,
                        lhist, xin_v, rows_v, din_v):
  c = lax.axis_index("c")
  s = lax.axis_index("s")
  wid = c * NS + s
  pltpu.sync_copy(z1_hbm, lhist)
  pltpu.sync_copy(dstr_hbm.at[wid], din_v)
  @pl.loop(0, RK)
  def _(k):
    cid = wid + k * NW
    @pl.when(cid < RNC)
    def _():
      pltpu.sync_copy(x_hbm.at[pl.ds(cid * RCH, RCH)], xin_v)
      pltpu.sync_copy(t1_hbm.at[xin_v], rows_v)
      pltpu.sync_copy(rows_v, hw1_hbm.at[pl.ds(cid * RCH, RCH)])
  ones16 = jnp.ones((16,), jnp.float32)
  @pl.loop(0, ENC)
  def _(i):
    for v in range(ECH // 16):
      iv = din_v[i, pl.ds(v * 16, 16)]
      plsc.addupdate_scatter(lhist, [iv], ones16)
  pltpu.sync_copy(lhist, hist_hbm.at[pl.ds(wid * N, N)])


_sc_gather_deg = pl.kernel(
    _sc_gather_deg_body,
    out_type=(jax.ShapeDtypeStruct((N, D), jnp.float32),
              jax.ShapeDtypeStruct((NW * N,), jnp.float32)),
    mesh=_mesh,
    compiler_params=_cp,
    scratch_types=[
        pltpu.VMEM((N,), jnp.float32),
        pltpu.VMEM((RCH,), jnp.int32),
        pltpu.VMEM((RCH, D), jnp.float32),
        pltpu.VMEM((ENC, ECH), jnp.int32),
    ],
)


def _sc_edge_body(g_hbm, srcr_hbm, dstr_hbm, z128_hbm, out_hbm,
                  acc, sidx_v, dbuf0, dbuf1, rows0, rows1, gsem, ssem):
  c = lax.axis_index("c")
  s = lax.axis_index("s")
  wid = c * NS + s
  r0 = s * RPT
  @pl.when(s < NS - 1)
  def _():
    pltpu.sync_copy(z128_hbm, acc.at[pl.ds(r0, RPT)])
  @pl.when(s == NS - 1)
  def _():
    pltpu.sync_copy(z128_hbm.at[pl.ds(0, RPT_LAST)],
                    acc.at[pl.ds(r0, RPT_LAST)])
  pltpu.sync_copy(srcr_hbm.at[wid], sidx_v)
  plsc.subcore_barrier()

  rows = (rows0, rows1)
  dbufs = (dbuf0, dbuf1)

  def _gather(i, b):
    return pltpu.make_async_copy(g_hbm.at[sidx_v.at[i]], rows[b], gsem.at[b])

  def _scatter(idx_ref, b):
    return pltpu.make_async_copy(rows[b], acc.at[idx_ref], ssem.at[b])

  def _chunk(i, k, dbuf, phase):
    slot = (phase + k) % NBUF
    nslot = (slot + 1) % NBUF
    _gather(i, slot).wait()
    _scatter(dbuf.at[k], slot).start(add=True)
    nxt = i + 1
    @pl.when(nxt < ENC)
    def _():
      @pl.when(nxt >= NBUF)
      def _():
        _scatter(dbuf.at[k], nslot).wait()
      _gather(nxt, nslot).start()

  _gather(0, 0).start()

  @pl.loop(0, (NGRP - 1) // 2)
  def _(t):
    for half in range(2):
      grp = t * 2 + half
      dbuf = dbufs[half]
      pltpu.sync_copy(dstr_hbm.at[wid, grp], dbuf)
      for k in range(GSZ):
        _chunk(grp * GSZ + k, k, dbuf, half * GSZ)

  grp = NGRP - 1
  pltpu.sync_copy(dstr_hbm.at[wid, grp], dbuf0)
  for k in range(GSZ):
    _chunk(grp * GSZ + k, k, dbuf0, 0)

  for i in range(ENC - NBUF, ENC):
    _scatter(dbuf0.at[GSZ - 1], i % NBUF).wait()
  plsc.subcore_barrier()
  @pl.when(s < NS - 1)
  def _():
    pltpu.sync_copy(acc.at[pl.ds(r0, RPT)],
                    out_hbm.at[pl.ds(c * N + r0, RPT)])
  @pl.when(s == NS - 1)
  def _():
    pltpu.sync_copy(acc.at[pl.ds(r0, RPT_LAST)],
                    out_hbm.at[pl.ds(c * N + r0, RPT_LAST)])


_sc_edge = pl.kernel(
    _sc_edge_body,
    out_type=jax.ShapeDtypeStruct((NC * N, D), jnp.float32),
    mesh=_mesh,
    scratch_types=[
        pltpu.VMEM_SHARED((N, D), jnp.float32),
        pltpu.VMEM((ENC, ECH), jnp.int32),
        pltpu.VMEM((GSZ, ECH), jnp.int32),
        pltpu.VMEM((GSZ, ECH), jnp.int32),
        pltpu.VMEM((ECH, D), jnp.float32),
        pltpu.VMEM((ECH, D), jnp.float32),
        pltpu.SemaphoreType.DMA((NBUF,)),
        pltpu.SemaphoreType.DMA((NBUF,)),
    ],
)


def _tc_scale_body(h_ref, w1_ref, hist_ref, g1_ref, dinv_ref):
  deg = 1.0 + jnp.sum(hist_ref[...], axis=0)[:, None]
  dinv = lax.rsqrt(deg)
  dinv_ref[...] = dinv
  hw1 = jnp.dot(h_ref[...], w1_ref[...], preferred_element_type=jnp.float32)
  g1_ref[...] = hw1 * dinv


_tc_scale = pl.pallas_call(
    _tc_scale_body,
    out_shape=(jax.ShapeDtypeStruct((N, D), jnp.float32),
               jax.ShapeDtypeStruct((N, 1), jnp.float32)),
)


def _tc_layer2_body(s1_ref, g1_ref, dinv_ref, b1_ref, w2_ref, g2_ref):
  dinv = dinv_ref[...]
  h1 = jnp.maximum(
      dinv * (s1_ref[0:N] + s1_ref[N:2 * N] + g1_ref[...]) + b1_ref[...], 0.0)
  hw2 = jnp.dot(h1, w2_ref[...], preferred_element_type=jnp.float32)
  g2_ref[...] = dinv * hw2


_tc_layer2 = pl.pallas_call(
    _tc_layer2_body,
    out_shape=jax.ShapeDtypeStruct((N, D), jnp.float32),
)


def _tc_final_body(s2_ref, g2_ref, dinv_ref, b2_ref, batch_ref,
                   wl1_ref, bl1_ref, wl2_ref, bl2_ref, o_ref):
  dinv = dinv_ref[...]
  h2 = dinv * (s2_ref[0:N] + s2_ref[N:2 * N] + g2_ref[...]) + b2_ref[...]
  iot = lax.broadcasted_iota(jnp.int32, (B, N), 0)
  bm = (jnp.broadcast_to(batch_ref[...], (B, N)) == iot).astype(jnp.float32)
  ssum = jnp.dot(bm, h2, preferred_element_type=jnp.float32)
  cnt = jnp.sum(bm, axis=1, keepdims=True)
  pooled = ssum / jnp.maximum(cnt, 1.0)
  z = jnp.maximum(
      jnp.dot(pooled, wl1_ref[...], preferred_element_type=jnp.float32)
      + bl1_ref[...], 0.0)
  t = (jnp.dot(z, wl2_ref[...], preferred_element_type=jnp.float32)
       + bl2_ref[...])
  o_ref[...] = 1.0 / (1.0 + jnp.exp(-t))


_tc_final = pl.pallas_call(
    _tc_final_body,
    out_shape=jax.ShapeDtypeStruct((B, 1), jnp.float32),
)


def kernel(x, edge_index, batch, emb_table, W1, b1, W2, b2, Wl1, bl1, Wl2, bl2):
  x = x.astype(jnp.int32)
  src = edge_index[0].astype(jnp.int32).reshape(NW, ENC, ECH)
  dst = edge_index[1].astype(jnp.int32).reshape(NW, ENC, ECH)
  z128 = jnp.zeros((RPT, D), jnp.float32)
  z1d = jnp.zeros((N,), jnp.float32)

  h0, hist = _sc_gather_deg(emb_table, x, dst, z1d)
  g1, dinv = _tc_scale(h0, W1, hist.reshape(NW, N))
  dst_g = dst.reshape(NW, NGRP, GSZ, ECH)
  s1 = _sc_edge(g1, src, dst_g, z128)
  g2 = _tc_layer2(s1, g1, dinv, b1.reshape(1, D), W2)
  s2 = _sc_edge(g2, src, dst_g, z128)
  out = _tc_final(s2, g2, dinv, b2.reshape(1, D),
                  batch.astype(jnp.int32).reshape(1, N),
                  Wl1, bl1.reshape(1, LD), Wl2, bl2.reshape(1, 1))
  return out


# speedup vs baseline: 2.7043x; 1.0152x over previous
import dataclasses

import jax
import jax.numpy as jnp
from jax import lax
from jax.experimental import pallas as pl
from jax.experimental.pallas import tpu as pltpu
from jax.experimental.pallas import tpu_sc as plsc

N = 10000
E = 320000
VOCAB = 10000
D = 128
B = 16
LD = 64

NC = 2
NS = 16
NW = NC * NS

EPW = E // NW
ECH = 80
ENC = EPW // ECH
NBUF = 2
GSZ = 25
NGRP = ENC // GSZ

RCH = 40
RNC = N // RCH
RK = -(-RNC // NW)

RPT = 632
RPT_LAST = N - (NS - 1) * RPT

_mesh = plsc.VectorSubcoreMesh(core_axis_name="c", subcore_axis_name="s")

_cp = pltpu.CompilerParams()
if "needs_layout_passes" in pltpu.CompilerParams.__dataclass_fields__:
  _cp = dataclasses.replace(_cp, needs_layout_passes=False)


def _sc_gather_deg_body(t1_hbm, x_hbm, dstr_hbm, z1_hbm,
                        hw1_hbm, hist_hbm,
                        lhist, xin0, xin1, rows0, rows1, din_v,
                        zsem, dnsem, isem, gsem, stsem):
  c = lax.axis_index("c")
  s = lax.axis_index("s")
  wid = c * NS + s
  pltpu.make_async_copy(z1_hbm, lhist, zsem).start()
  pltpu.make_async_copy(dstr_hbm.at[wid], din_v, dnsem).start()

  xins = (xin0, xin1)
  rows = (rows0, rows1)

  def _cid(k):
    return wid + k * NW

  def _idx(k, b):
    return pltpu.make_async_copy(x_hbm.at[pl.ds(_cid(k) * RCH, RCH)],
                                 xins[b], isem.at[b])

  def _grow(k, b):
    return pltpu.make_async_copy(t1_hbm.at[xins[b]], rows[b], gsem.at[b])

  def _store(k, b):
    return pltpu.make_async_copy(rows[b], hw1_hbm.at[pl.ds(_cid(k) * RCH, RCH)],
                                 stsem.at[b])

  @pl.when(_cid(0) < RNC)
  def _():
    _idx(0, 0).start()
  for k in range(RK):
    b = k % 2
    @pl.when(_cid(k) < RNC)
    def _():
      if k >= 2:
        _store(k - 2, b).wait()
      _idx(k, b).wait()
      _grow(k, b).start()
      if k + 1 < RK:
        @pl.when(_cid(k + 1) < RNC)
        def _():
          _idx(k + 1, (k + 1) % 2).start()
      _grow(k, b).wait()
      _store(k, b).start()
  for k in range(RK - 2, RK):
    @pl.when(_cid(k) < RNC)
    def _():
      _store(k, k % 2).wait()

  pltpu.make_async_copy(z1_hbm, lhist, zsem).wait()
  pltpu.make_async_copy(dstr_hbm.at[wid], din_v, dnsem).wait()
  ones16 = jnp.ones((16,), jnp.float32)
  @pl.loop(0, ENC)
  def _(i):
    for v in range(ECH // 16):
      iv = din_v[i, pl.ds(v * 16, 16)]
      plsc.addupdate_scatter(lhist, [iv], ones16)
  pltpu.sync_copy(lhist, hist_hbm.at[pl.ds(wid * N, N)])


_sc_gather_deg = pl.kernel(
    _sc_gather_deg_body,
    out_type=(jax.ShapeDtypeStruct((N, D), jnp.float32),
              jax.ShapeDtypeStruct((NW * N,), jnp.float32)),
    mesh=_mesh,
    compiler_params=_cp,
    scratch_types=[
        pltpu.VMEM((N,), jnp.float32),
        pltpu.VMEM((RCH,), jnp.int32),
        pltpu.VMEM((RCH,), jnp.int32),
        pltpu.VMEM((RCH, D), jnp.float32),
        pltpu.VMEM((RCH, D), jnp.float32),
        pltpu.VMEM((ENC, ECH), jnp.int32),
        pltpu.SemaphoreType.DMA,
        pltpu.SemaphoreType.DMA,
        pltpu.SemaphoreType.DMA((2,)),
        pltpu.SemaphoreType.DMA((2,)),
        pltpu.SemaphoreType.DMA((2,)),
    ],
)


def _sc_edge_body(g_hbm, srcr_hbm, dstr_hbm, z128_hbm, out_hbm,
                  acc, sidx_v, dbuf0, dbuf1, rows0, rows1, gsem, ssem):
  c = lax.axis_index("c")
  s = lax.axis_index("s")
  wid = c * NS + s
  r0 = s * RPT
  @pl.when(s < NS - 1)
  def _():
    pltpu.sync_copy(z128_hbm, acc.at[pl.ds(r0, RPT)])
  @pl.when(s == NS - 1)
  def _():
    pltpu.sync_copy(z128_hbm.at[pl.ds(0, RPT_LAST)],
                    acc.at[pl.ds(r0, RPT_LAST)])
  pltpu.sync_copy(srcr_hbm.at[wid], sidx_v)
  plsc.subcore_barrier()

  rows = (rows0, rows1)
  dbufs = (dbuf0, dbuf1)

  def _gather(i, b):
    return pltpu.make_async_copy(g_hbm.at[sidx_v.at[i]], rows[b], gsem.at[b])

  def _scatter(idx_ref, b):
    return pltpu.make_async_copy(rows[b], acc.at[idx_ref], ssem.at[b])

  def _chunk(i, k, dbuf, phase):
    slot = (phase + k) % NBUF
    nslot = (slot + 1) % NBUF
    _gather(i, slot).wait()
    _scatter(dbuf.at[k], slot).start(add=True)
    nxt = i + 1
    @pl.when(nxt < ENC)
    def _():
      @pl.when(nxt >= NBUF)
      def _():
        _scatter(dbuf.at[k], nslot).wait()
      _gather(nxt, nslot).start()

  _gather(0, 0).start()

  @pl.loop(0, (NGRP - 1) // 2)
  def _(t):
    for half in range(2):
      grp = t * 2 + half
      dbuf = dbufs[half]
      pltpu.sync_copy(dstr_hbm.at[wid, grp], dbuf)
      for k in range(GSZ):
        _chunk(grp * GSZ + k, k, dbuf, half * GSZ)

  grp = NGRP - 1
  pltpu.sync_copy(dstr_hbm.at[wid, grp], dbuf0)
  for k in range(GSZ):
    _chunk(grp * GSZ + k, k, dbuf0, 0)

  for i in range(ENC - NBUF, ENC):
    _scatter(dbuf0.at[GSZ - 1], i % NBUF).wait()
  plsc.subcore_barrier()
  @pl.when(s < NS - 1)
  def _():
    pltpu.sync_copy(acc.at[pl.ds(r0, RPT)],
                    out_hbm.at[pl.ds(c * N + r0, RPT)])
  @pl.when(s == NS - 1)
  def _():
    pltpu.sync_copy(acc.at[pl.ds(r0, RPT_LAST)],
                    out_hbm.at[pl.ds(c * N + r0, RPT_LAST)])


_sc_edge = pl.kernel(
    _sc_edge_body,
    out_type=jax.ShapeDtypeStruct((NC * N, D), jnp.float32),
    mesh=_mesh,
    scratch_types=[
        pltpu.VMEM_SHARED((N, D), jnp.float32),
        pltpu.VMEM((ENC, ECH), jnp.int32),
        pltpu.VMEM((GSZ, ECH), jnp.int32),
        pltpu.VMEM((GSZ, ECH), jnp.int32),
        pltpu.VMEM((ECH, D), jnp.float32),
        pltpu.VMEM((ECH, D), jnp.float32),
        pltpu.SemaphoreType.DMA((NBUF,)),
        pltpu.SemaphoreType.DMA((NBUF,)),
    ],
)


def _tc_scale_body(h_ref, w1_ref, hist_ref, g1_ref, dinv_ref):
  deg = 1.0 + jnp.sum(hist_ref[...], axis=0)[:, None]
  dinv = lax.rsqrt(deg)
  dinv_ref[...] = dinv
  hw1 = jnp.dot(h_ref[...], w1_ref[...], preferred_element_type=jnp.float32)
  g1_ref[...] = hw1 * dinv


_tc_scale = pl.pallas_call(
    _tc_scale_body,
    out_shape=(jax.ShapeDtypeStruct((N, D), jnp.float32),
               jax.ShapeDtypeStruct((N, 1), jnp.float32)),
)


def _tc_layer2_body(s1_ref, g1_ref, dinv_ref, b1_ref, w2_ref, g2_ref):
  dinv = dinv_ref[...]
  h1 = jnp.maximum(
      dinv * (s1_ref[0:N] + s1_ref[N:2 * N] + g1_ref[...]) + b1_ref[...], 0.0)
  hw2 = jnp.dot(h1, w2_ref[...], preferred_element_type=jnp.float32)
  g2_ref[...] = dinv * hw2


_tc_layer2 = pl.pallas_call(
    _tc_layer2_body,
    out_shape=jax.ShapeDtypeStruct((N, D), jnp.float32),
)


def _tc_final_body(s2_ref, g2_ref, dinv_ref, b2_ref, batch_ref,
                   wl1_ref, bl1_ref, wl2_ref, bl2_ref, o_ref):
  dinv = dinv_ref[...]
  h2 = dinv * (s2_ref[0:N] + s2_ref[N:2 * N] + g2_ref[...]) + b2_ref[...]
  iot = lax.broadcasted_iota(jnp.int32, (B, N), 0)
  bm = (jnp.broadcast_to(batch_ref[...], (B, N)) == iot).astype(jnp.float32)
  ssum = jnp.dot(bm, h2, preferred_element_type=jnp.float32)
  cnt = jnp.sum(bm, axis=1, keepdims=True)
  pooled = ssum / jnp.maximum(cnt, 1.0)
  z = jnp.maximum(
      jnp.dot(pooled, wl1_ref[...], preferred_element_type=jnp.float32)
      + bl1_ref[...], 0.0)
  t = (jnp.dot(z, wl2_ref[...], preferred_element_type=jnp.float32)
       + bl2_ref[...])
  o_ref[...] = 1.0 / (1.0 + jnp.exp(-t))


_tc_final = pl.pallas_call(
    _tc_final_body,
    out_shape=jax.ShapeDtypeStruct((B, 1), jnp.float32),
)


def kernel(x, edge_index, batch, emb_table, W1, b1, W2, b2, Wl1, bl1, Wl2, bl2):
  x = x.astype(jnp.int32)
  src = edge_index[0].astype(jnp.int32).reshape(NW, ENC, ECH)
  dst = edge_index[1].astype(jnp.int32).reshape(NW, ENC, ECH)
  z128 = jnp.zeros((RPT, D), jnp.float32)
  z1d = jnp.zeros((N,), jnp.float32)

  h0, hist = _sc_gather_deg(emb_table, x, dst, z1d)
  g1, dinv = _tc_scale(h0, W1, hist.reshape(NW, N))
  dst_g = dst.reshape(NW, NGRP, GSZ, ECH)
  s1 = _sc_edge(g1, src, dst_g, z128)
  g2 = _tc_layer2(s1, g1, dinv, b1.reshape(1, D), W2)
  s2 = _sc_edge(g2, src, dst_g, z128)
  out = _tc_final(s2, g2, dinv, b2.reshape(1, D),
                  batch.astype(jnp.int32).reshape(1, N),
                  Wl1, bl1.reshape(1, LD), Wl2, bl2.reshape(1, 1))
  return out
